# Initial kernel scaffold; baseline (speedup 1.0000x reference)
import functools

import jax
import jax.numpy as jnp
from jax import lax
from jax.experimental import pallas as pl
from jax.experimental.pallas import tpu as pltpu
from jax.experimental.pallas import tpu_sc as plsc

B = 16384
LQ = 20
LD = 50
LT = LQ + LD
D = 64
H = 128

NC = 2
NS = 16
NW = NC * NS
ROWS_W = B // NW
R = 8
CHUNK = R * LT
NCH = ROWS_W // R


def _sc_pool_body(ids_hbm, table_hbm, qout, dout,
                  idx0, idx1, rows0, rows1, oq, od, sem0, sem1):
    wid = lax.axis_index("s") * NC + lax.axis_index("c")
    row0 = wid * ROWS_W
    i0 = row0 * LT

    def start(c, idx_v, rows_v, sem):
        pltpu.sync_copy(ids_hbm.at[pl.ds(i0 + c * CHUNK, CHUNK)], idx_v)
        pltpu.async_copy(table_hbm.at[idx_v], rows_v, sem)

    def wait(idx_v, rows_v, sem):
        pltpu.make_async_copy(table_hbm.at[idx_v], rows_v, sem).wait()

    def compute(c, rows_v):
        for r in range(R):
            rb = r * LT

            def body5(j5, acc):
                accs = list(acc)
                b0 = rb + j5 * 5
                for jj in range(5):
                    for k in range(4):
                        accs[k] = accs[k] + rows_v[b0 + jj, pl.ds(k * 16, 16)]
                return tuple(accs)

            z = jnp.zeros((16,), jnp.float32)
            qa = lax.fori_loop(0, LQ // 5, body5, (z, z, z, z))
            da = lax.fori_loop(LQ // 5, LT // 5, body5, (z, z, z, z))
            for k in range(4):
                oq[r, pl.ds(k * 16, 16)] = qa[k]
                od[r, pl.ds(k * 16, 16)] = da[k]
        pltpu.sync_copy(oq, qout.at[pl.ds(row0 + c * R, R)])
        pltpu.sync_copy(od, dout.at[pl.ds(row0 + c * R, R)])

    start(0, idx0, rows0, sem0)

    def outer(i, carry):
        c0 = 2 * i
        start(c0 + 1, idx1, rows1, sem1)
        wait(idx0, rows0, sem0)
        compute(c0, rows0)

        @pl.when(i < NCH // 2 - 1)
        def _():
            start(c0 + 2, idx0, rows0, sem0)

        wait(idx1, rows1, sem1)
        compute(c0 + 1, rows1)
        return carry

    lax.fori_loop(0, NCH // 2, outer, 0)


def _make_sc_pool():
    return functools.partial(
        pl.kernel,
        out_type=(jax.ShapeDtypeStruct((B, D), jnp.float32),
                  jax.ShapeDtypeStruct((B, D), jnp.float32)),
        mesh=plsc.VectorSubcoreMesh(core_axis_name="c", subcore_axis_name="s"),
        scratch_types=[
            pltpu.VMEM((CHUNK,), jnp.int32),
            pltpu.VMEM((CHUNK,), jnp.int32),
            pltpu.VMEM((CHUNK, D), jnp.float32),
            pltpu.VMEM((CHUNK, D), jnp.float32),
            pltpu.VMEM((R, D), jnp.float32),
            pltpu.VMEM((R, D), jnp.float32),
            pltpu.SemaphoreType.DMA,
            pltpu.SemaphoreType.DMA,
        ],
    )(_sc_pool_body)


_sc_pool = _make_sc_pool()


def _mlp_body(qs_ref, ds_ref, qid_ref, did_ref, w1_ref, b1_ref, w2_ref,
              b2_ref, out_ref):
    qc = jnp.maximum(
        jnp.sum((qid_ref[...] != 0).astype(jnp.float32), axis=1,
                keepdims=True), 1.0)
    dc = jnp.maximum(
        jnp.sum((did_ref[...] != 0).astype(jnp.float32), axis=1,
                keepdims=True), 1.0)
    q = qs_ref[...] / qc
    d = ds_ref[...] / dc
    diff = jnp.abs(q - d)
    w1 = w1_ref[...]
    h = (jnp.dot(q, w1[0:D], preferred_element_type=jnp.float32)
         + jnp.dot(d, w1[D:2 * D], preferred_element_type=jnp.float32)
         + jnp.dot(diff, w1[2 * D:3 * D], preferred_element_type=jnp.float32)
         + b1_ref[...])
    h = jnp.maximum(h, 0.0)
    out_ref[...] = jnp.sum(h * w2_ref[...], axis=1) + b2_ref[0]


_BC = 2048


def _mlp(qsum, dsum, qi, di, W1, b1, w2row, b2):
    return pl.pallas_call(
        _mlp_body,
        grid=(B // _BC,),
        in_specs=[
            pl.BlockSpec((_BC, D), lambda i: (i, 0)),
            pl.BlockSpec((_BC, D), lambda i: (i, 0)),
            pl.BlockSpec((_BC, LQ), lambda i: (i, 0)),
            pl.BlockSpec((_BC, LD), lambda i: (i, 0)),
            pl.BlockSpec((3 * D, H), lambda i: (0, 0)),
            pl.BlockSpec((H,), lambda i: (0,)),
            pl.BlockSpec((1, H), lambda i: (0, 0)),
            pl.BlockSpec((1,), lambda i: (0,)),
        ],
        out_specs=pl.BlockSpec((_BC,), lambda i: (i,)),
        out_shape=jax.ShapeDtypeStruct((B,), jnp.float32),
    )(qsum, dsum, qi, di, W1, b1, w2row, b2)


def kernel(query_ids, dish_ids, table, W1, b1, W2, b2):
    qi = query_ids.astype(jnp.int32)
    di = dish_ids.astype(jnp.int32)
    ids_flat = jnp.concatenate([qi, di], axis=1).reshape(-1)
    qsum, dsum = _sc_pool(ids_flat, table)
    return _mlp(qsum, dsum, qi, di, W1, b1, W2.reshape(1, H),
                b2.reshape(1,))


# speedup vs baseline: 2.7501x; 2.7501x over previous
import functools

import jax
import jax.numpy as jnp
from jax import lax
from jax.experimental import pallas as pl
from jax.experimental.pallas import tpu as pltpu
from jax.experimental.pallas import tpu_sc as plsc

B = 16384
LQ = 20
LD = 50
LT = LQ + LD
D = 64
H = 128

NC = 2
NS = 16
NW = NC * NS
ROWS_W = B // NW
R = 8
CHUNK = R * LT
NCH = ROWS_W // R


def _sc_pool_body(ids_hbm, table_hbm, qout, dout,
                  idx0, idx1, rows0, rows1, oq, od, sem0, sem1):
    wid = lax.axis_index("s") * NC + lax.axis_index("c")
    row0 = wid * ROWS_W
    i0 = row0 * LT

    def start(c, idx_v, rows_v, sem):
        pltpu.sync_copy(ids_hbm.at[pl.ds(i0 + c * CHUNK, CHUNK)], idx_v)
        pltpu.async_copy(table_hbm.at[idx_v], rows_v, sem)

    def wait(idx_v, rows_v, sem):
        pltpu.make_async_copy(table_hbm.at[idx_v], rows_v, sem).wait()

    def compute(c, rows_v):
        for r in range(R):
            rb = r * LT

            def body5(j5, acc):
                accs = list(acc)
                b0 = rb + j5 * 5
                for jj in range(5):
                    for k in range(4):
                        accs[k] = accs[k] + rows_v[b0 + jj, pl.ds(k * 16, 16)]
                return tuple(accs)

            z = jnp.zeros((16,), jnp.float32)
            qa = lax.fori_loop(0, LQ // 5, body5, (z, z, z, z))
            da = lax.fori_loop(LQ // 5, LT // 5, body5, (z, z, z, z))
            for k in range(4):
                oq[r, pl.ds(k * 16, 16)] = qa[k]
                od[r, pl.ds(k * 16, 16)] = da[k]
        pltpu.sync_copy(oq, qout.at[pl.ds(row0 + c * R, R)])
        pltpu.sync_copy(od, dout.at[pl.ds(row0 + c * R, R)])

    start(0, idx0, rows0, sem0)

    def outer(i, carry):
        c0 = 2 * i
        start(c0 + 1, idx1, rows1, sem1)
        wait(idx0, rows0, sem0)
        compute(c0, rows0)

        @pl.when(i < NCH // 2 - 1)
        def _():
            start(c0 + 2, idx0, rows0, sem0)

        wait(idx1, rows1, sem1)
        compute(c0 + 1, rows1)
        return carry

    lax.fori_loop(0, NCH // 2, outer, 0)


def _make_sc_pool():
    return functools.partial(
        pl.kernel,
        out_type=(jax.ShapeDtypeStruct((B, D), jnp.float32),
                  jax.ShapeDtypeStruct((B, D), jnp.float32)),
        mesh=plsc.VectorSubcoreMesh(core_axis_name="c", subcore_axis_name="s"),
        compiler_params=pltpu.CompilerParams(use_tc_tiling_on_sc=False),
        scratch_types=[
            pltpu.VMEM((CHUNK,), jnp.int32),
            pltpu.VMEM((CHUNK,), jnp.int32),
            pltpu.VMEM((CHUNK, D), jnp.float32),
            pltpu.VMEM((CHUNK, D), jnp.float32),
            pltpu.VMEM((R, D), jnp.float32),
            pltpu.VMEM((R, D), jnp.float32),
            pltpu.SemaphoreType.DMA,
            pltpu.SemaphoreType.DMA,
        ],
    )(_sc_pool_body)


_sc_pool = _make_sc_pool()


def _mlp_body(qs_ref, ds_ref, qid_ref, did_ref, w1_ref, b1_ref, w2_ref,
              b2_ref, out_ref):
    qc = jnp.maximum(
        jnp.sum((qid_ref[...] != 0).astype(jnp.float32), axis=1,
                keepdims=True), 1.0)
    dc = jnp.maximum(
        jnp.sum((did_ref[...] != 0).astype(jnp.float32), axis=1,
                keepdims=True), 1.0)
    q = qs_ref[...] / qc
    d = ds_ref[...] / dc
    diff = jnp.abs(q - d)
    w1 = w1_ref[...]
    h = (jnp.dot(q, w1[0:D], preferred_element_type=jnp.float32)
         + jnp.dot(d, w1[D:2 * D], preferred_element_type=jnp.float32)
         + jnp.dot(diff, w1[2 * D:3 * D], preferred_element_type=jnp.float32)
         + b1_ref[...])
    h = jnp.maximum(h, 0.0)
    out_ref[...] = jnp.sum(h * w2_ref[...], axis=1) + b2_ref[0]


_BC = 2048


def _mlp(qsum, dsum, qi, di, W1, b1, w2row, b2):
    return pl.pallas_call(
        _mlp_body,
        grid=(B // _BC,),
        in_specs=[
            pl.BlockSpec((_BC, D), lambda i: (i, 0)),
            pl.BlockSpec((_BC, D), lambda i: (i, 0)),
            pl.BlockSpec((_BC, LQ), lambda i: (i, 0)),
            pl.BlockSpec((_BC, LD), lambda i: (i, 0)),
            pl.BlockSpec((3 * D, H), lambda i: (0, 0)),
            pl.BlockSpec((H,), lambda i: (0,)),
            pl.BlockSpec((1, H), lambda i: (0, 0)),
            pl.BlockSpec((1,), lambda i: (0,)),
        ],
        out_specs=pl.BlockSpec((_BC,), lambda i: (i,)),
        out_shape=jax.ShapeDtypeStruct((B,), jnp.float32),
    )(qsum, dsum, qi, di, W1, b1, w2row, b2)


def kernel(query_ids, dish_ids, table, W1, b1, W2, b2):
    qi = query_ids.astype(jnp.int32)
    di = dish_ids.astype(jnp.int32)
    ids_flat = jnp.concatenate([qi, di], axis=1).reshape(-1)
    qsum, dsum = _sc_pool(ids_flat, table)
    return _mlp(qsum, dsum, qi, di, W1, b1, W2.reshape(1, H),
                b2.reshape(1,))
